# baseline (speedup 1.0000x reference)
import functools

import jax
import jax.numpy as jnp
from jax import lax
from jax.experimental import pallas as pl
from jax.experimental.pallas import tpu as pltpu
from jax.experimental.pallas import tpu_sc as plsc

NC = 2
NS = 16
NW = NC * NS
L = 16

_SC_PARAMS = pltpu.CompilerParams(needs_layout_passes=False)


def _noisy_or_kernel(batch: int, rules_per_ex: int, gpad: int, wpad: int,
                     pad_id: int):
    ex_per_w = batch // NW
    chunk_ex = 32
    groups = chunk_ex // L
    n_chunks = ex_per_w // chunk_ex
    chunk_elems = chunk_ex * rules_per_ex
    mesh = plsc.VectorSubcoreMesh(core_axis_name="c", subcore_axis_name="s")

    @functools.partial(
        pl.kernel,
        out_type=jax.ShapeDtypeStruct((batch,), jnp.float32),
        mesh=mesh,
        scratch_types=[
            pltpu.VMEM((gpad,), jnp.int32),
            pltpu.VMEM((wpad,), jnp.float32),
            pltpu.VMEM((chunk_elems,), jnp.int32),
            pltpu.VMEM((chunk_elems,), jnp.int32),
            pltpu.VMEM((chunk_elems,), jnp.int32),
            pltpu.VMEM((ex_per_w,), jnp.float32),
            pltpu.SemaphoreType.DMA,
            pltpu.SemaphoreType.DMA,
            pltpu.SemaphoreType.DMA,
            pltpu.SemaphoreType.DMA,
        ],
        compiler_params=_SC_PARAMS,
    )
    def agg(rules_hbm, g2l_hbm, w_hbm, out_hbm, g2l_v, w_v, rul_a, rul_b,
            rul_c, out_v, tab_sem, sem_a, sem_b, sem_c):
        wid = lax.axis_index("s") * NC + lax.axis_index("c")
        ex_base = wid * ex_per_w
        bufs = (rul_a, rul_b, rul_c)
        sems = (sem_a, sem_b, sem_c)
        nbuf = len(bufs)

        def start(c, buf, sem):
            return pltpu.async_copy(
                rules_hbm.at[
                    pl.ds((ex_base + c * chunk_ex) * rules_per_ex, chunk_elems)
                ],
                buf, sem,
            )

        g2l_cp = pltpu.async_copy(g2l_hbm, g2l_v, tab_sem)
        pend = [start(c, bufs[c], sems[c]) for c in range(nbuf - 1)]
        pend.append(None)
        pltpu.sync_copy(w_hbm, w_v)
        g2l_cp.wait()

        colbases = [
            lax.iota(jnp.int32, L) * rules_per_ex + g * L * rules_per_ex
            for g in range(groups)
        ]

        for c in range(n_chunks):
            if c + nbuf - 1 < n_chunks:
                k = (c + nbuf - 1) % nbuf
                pend[k] = start(c + nbuf - 1, bufs[k], sems[k])
            pend[c % nbuf].wait()
            rules_v = bufs[c % nbuf]

            def step(r, accs):
                new = []
                for g in range(groups):
                    idx = plsc.load_gather(rules_v, [colbases[g] + r])
                    local = plsc.load_gather(g2l_v, [idx])
                    w = plsc.load_gather(w_v, [local])
                    e = jnp.where(local == pad_id, 0.0, jnp.exp(w))
                    new.append(accs[g] * (1.0 + e))
                return tuple(new)

            accs = lax.fori_loop(
                0, rules_per_ex, step,
                tuple(jnp.ones((L,), jnp.float32) for _ in range(groups)),
                unroll=8,
            )
            for g in range(groups):
                out_v[pl.ds(c * chunk_ex + g * L, L)] = jnp.clip(
                    1.0 - 1.0 / accs[g], 1e-4, 0.99999
                )

        pltpu.sync_copy(out_v, out_hbm.at[pl.ds(ex_base, ex_per_w)])

    return agg


def kernel(rules, g2l, W):
    batch, rules_per_ex = rules.shape
    g = g2l.shape[0]
    pad_id = W.shape[0] - 1

    gpad = -(-g // L) * L
    g2l_pad = jnp.concatenate([g2l, jnp.full((gpad - g,), pad_id, jnp.int32)])
    wpad = -(-W.shape[0] // 8) * 8
    w_flat = jnp.concatenate(
        [W.reshape(-1), jnp.zeros((wpad - W.shape[0],), jnp.float32)]
    )

    out = _noisy_or_kernel(batch, rules_per_ex, gpad, wpad, pad_id)(
        rules.reshape(-1), g2l_pad, w_flat
    )
    return out.reshape(batch, 1)


# speedup vs baseline: 1.0181x; 1.0181x over previous
import functools

import jax
import jax.numpy as jnp
from jax import lax
from jax.experimental import pallas as pl
from jax.experimental.pallas import tpu as pltpu
from jax.experimental.pallas import tpu_sc as plsc

NC = 2
NS = 16
NW = NC * NS
L = 16

_SC_PARAMS = pltpu.CompilerParams(needs_layout_passes=False)


def _build_table_kernel(gpad: int, wpad: int, pad_id: int):
    g_per_w = gpad // NW
    nvec = g_per_w // L
    mesh = plsc.VectorSubcoreMesh(core_axis_name="c", subcore_axis_name="s")

    @functools.partial(
        pl.kernel,
        out_type=jax.ShapeDtypeStruct((gpad,), jnp.float32),
        mesh=mesh,
        scratch_types=[
            pltpu.VMEM((g_per_w,), jnp.int32),
            pltpu.VMEM((wpad,), jnp.float32),
            pltpu.VMEM((g_per_w,), jnp.float32),
        ],
        compiler_params=_SC_PARAMS,
    )
    def build(g2l_hbm, w_hbm, f_hbm, g2l_v, w_v, f_v):
        wid = lax.axis_index("s") * NC + lax.axis_index("c")
        base = wid * g_per_w
        pltpu.sync_copy(g2l_hbm.at[pl.ds(base, g_per_w)], g2l_v)
        pltpu.sync_copy(w_hbm, w_v)

        @pl.loop(0, nvec)
        def _vec(v):
            local = g2l_v[pl.ds(v * L, L)]
            w = plsc.load_gather(w_v, [local])
            f = jnp.where(local == pad_id, 1.0, 1.0 / (1.0 + jnp.exp(w)))
            f_v[pl.ds(v * L, L)] = f

        pltpu.sync_copy(f_v, f_hbm.at[pl.ds(base, g_per_w)])

    return build


def _aggregate_kernel(batch: int, rules_per_ex: int, gpad: int):
    ex_per_w = batch // NW
    chunk_ex = 64
    groups = chunk_ex // L
    n_chunks = ex_per_w // chunk_ex
    chunk_elems = chunk_ex * rules_per_ex
    mesh = plsc.VectorSubcoreMesh(core_axis_name="c", subcore_axis_name="s")

    @functools.partial(
        pl.kernel,
        out_type=jax.ShapeDtypeStruct((batch,), jnp.float32),
        mesh=mesh,
        scratch_types=[
            pltpu.VMEM((gpad,), jnp.float32),
            pltpu.VMEM((chunk_elems,), jnp.int32),
            pltpu.VMEM((chunk_elems,), jnp.int32),
            pltpu.VMEM((ex_per_w,), jnp.float32),
            pltpu.SemaphoreType.DMA,
            pltpu.SemaphoreType.DMA,
            pltpu.SemaphoreType.DMA,
        ],
        compiler_params=_SC_PARAMS,
    )
    def agg(rules_hbm, f_hbm, out_hbm, f_v, rul_a, rul_b, out_v,
            f_sem, sem_a, sem_b):
        wid = lax.axis_index("s") * NC + lax.axis_index("c")
        ex_base = wid * ex_per_w
        bufs = (rul_a, rul_b)
        sems = (sem_a, sem_b)

        def start(c, buf, sem):
            return pltpu.async_copy(
                rules_hbm.at[
                    pl.ds((ex_base + c * chunk_ex) * rules_per_ex, chunk_elems)
                ],
                buf, sem,
            )

        f_cp = pltpu.async_copy(f_hbm, f_v, f_sem)
        pending = start(0, bufs[0], sems[0])
        f_cp.wait()

        colbases = [
            lax.iota(jnp.int32, L) * rules_per_ex + g * L * rules_per_ex
            for g in range(groups)
        ]

        for c in range(n_chunks):
            nxt = None
            if c + 1 < n_chunks:
                nxt = start(c + 1, bufs[(c + 1) % 2], sems[(c + 1) % 2])
            pending.wait()
            rules_v = bufs[c % 2]

            def step(r, accs):
                new = []
                for g in range(groups):
                    idx = plsc.load_gather(rules_v, [colbases[g] + r])
                    f = plsc.load_gather(f_v, [idx])
                    new.append(accs[g] * f)
                return tuple(new)

            accs = lax.fori_loop(
                0, rules_per_ex, step,
                tuple(jnp.ones((L,), jnp.float32) for _ in range(groups)),
                unroll=2,
            )
            for g in range(groups):
                out_v[pl.ds(c * chunk_ex + g * L, L)] = jnp.clip(
                    1.0 - accs[g], 1e-4, 0.99999
                )
            pending = nxt

        pltpu.sync_copy(out_v, out_hbm.at[pl.ds(ex_base, ex_per_w)])

    return agg


def kernel(rules, g2l, W):
    batch, rules_per_ex = rules.shape
    g = g2l.shape[0]
    pad_id = W.shape[0] - 1

    g_per_w = -(-g // (NW * L)) * L
    gpad = g_per_w * NW
    g2l_pad = jnp.concatenate(
        [g2l, jnp.full((gpad - g,), pad_id, jnp.int32)]
    )
    wpad = -(-W.shape[0] // 8) * 8
    w_flat = jnp.concatenate(
        [W.reshape(-1), jnp.zeros((wpad - W.shape[0],), jnp.float32)]
    )

    f_table = _build_table_kernel(gpad, wpad, pad_id)(g2l_pad, w_flat)
    out = _aggregate_kernel(batch, rules_per_ex, gpad)(
        rules.reshape(-1), f_table
    )
    return out.reshape(batch, 1)


# speedup vs baseline: 1.0255x; 1.0073x over previous
import functools

import jax
import jax.numpy as jnp
from jax import lax
from jax.experimental import pallas as pl
from jax.experimental.pallas import tpu as pltpu
from jax.experimental.pallas import tpu_sc as plsc

NC = 2
NS = 16
NW = NC * NS
L = 16

_SC_PARAMS = pltpu.CompilerParams(needs_layout_passes=False)


def _build_table_kernel(gpad: int, wpad: int, pad_id: int):
    g_per_w = gpad // NW
    nvec = g_per_w // L
    mesh = plsc.VectorSubcoreMesh(core_axis_name="c", subcore_axis_name="s")

    @functools.partial(
        pl.kernel,
        out_type=jax.ShapeDtypeStruct((gpad,), jnp.float32),
        mesh=mesh,
        scratch_types=[
            pltpu.VMEM((g_per_w,), jnp.int32),
            pltpu.VMEM((wpad,), jnp.float32),
            pltpu.VMEM((g_per_w,), jnp.float32),
        ],
        compiler_params=_SC_PARAMS,
    )
    def build(g2l_hbm, w_hbm, f_hbm, g2l_v, w_v, f_v):
        wid = lax.axis_index("s") * NC + lax.axis_index("c")
        base = wid * g_per_w
        pltpu.sync_copy(g2l_hbm.at[pl.ds(base, g_per_w)], g2l_v)
        pltpu.sync_copy(w_hbm, w_v)

        @pl.loop(0, nvec)
        def _vec(v):
            local = g2l_v[pl.ds(v * L, L)]
            w = plsc.load_gather(w_v, [local])
            f = jnp.where(local == pad_id, 1.0, 1.0 / (1.0 + jnp.exp(w)))
            f_v[pl.ds(v * L, L)] = f

        pltpu.sync_copy(f_v, f_hbm.at[pl.ds(base, g_per_w)])

    return build


def _aggregate_kernel(batch: int, rules_per_ex: int, gpad: int):
    ex_per_w = batch // NW
    chunk_ex = 64
    groups = chunk_ex // L
    n_chunks = ex_per_w // chunk_ex
    chunk_elems = chunk_ex * rules_per_ex
    mesh = plsc.VectorSubcoreMesh(core_axis_name="c", subcore_axis_name="s")

    @functools.partial(
        pl.kernel,
        out_type=jax.ShapeDtypeStruct((batch,), jnp.float32),
        mesh=mesh,
        scratch_types=[
            pltpu.VMEM((gpad,), jnp.float32),
            pltpu.VMEM((chunk_elems,), jnp.int32),
            pltpu.VMEM((chunk_elems,), jnp.int32),
            pltpu.VMEM((ex_per_w,), jnp.float32),
            pltpu.SemaphoreType.DMA,
            pltpu.SemaphoreType.DMA,
            pltpu.SemaphoreType.DMA,
        ],
        compiler_params=_SC_PARAMS,
    )
    def agg(rules_hbm, f_hbm, out_hbm, f_v, rul_a, rul_b, out_v,
            f_sem, sem_a, sem_b):
        wid = lax.axis_index("s") * NC + lax.axis_index("c")
        ex_base = wid * ex_per_w
        bufs = (rul_a, rul_b)
        sems = (sem_a, sem_b)

        def start(c, buf, sem):
            return pltpu.async_copy(
                rules_hbm.at[
                    pl.ds((ex_base + c * chunk_ex) * rules_per_ex, chunk_elems)
                ],
                buf, sem,
            )

        f_cp = pltpu.async_copy(f_hbm, f_v, f_sem)
        pending = start(0, bufs[0], sems[0])
        f_cp.wait()

        colbases = [
            lax.iota(jnp.int32, L) * rules_per_ex + g * L * rules_per_ex
            for g in range(groups)
        ]

        for c in range(n_chunks):
            nxt = None
            if c + 1 < n_chunks:
                nxt = start(c + 1, bufs[(c + 1) % 2], sems[(c + 1) % 2])
            pending.wait()
            rules_v = bufs[c % 2]

            def step(r, accs):
                new = []
                for g in range(groups):
                    idx = plsc.load_gather(rules_v, [colbases[g] + r])
                    f = plsc.load_gather(f_v, [idx])
                    new.append(accs[g] * f)
                return tuple(new)

            accs = lax.fori_loop(
                0, rules_per_ex, step,
                tuple(jnp.ones((L,), jnp.float32) for _ in range(groups)),
                unroll=4,
            )
            for g in range(groups):
                out_v[pl.ds(c * chunk_ex + g * L, L)] = jnp.clip(
                    1.0 - accs[g], 1e-4, 0.99999
                )
            pending = nxt

        pltpu.sync_copy(out_v, out_hbm.at[pl.ds(ex_base, ex_per_w)])

    return agg


def kernel(rules, g2l, W):
    batch, rules_per_ex = rules.shape
    g = g2l.shape[0]
    pad_id = W.shape[0] - 1

    g_per_w = -(-g // (NW * L)) * L
    gpad = g_per_w * NW
    g2l_pad = jnp.concatenate(
        [g2l, jnp.full((gpad - g,), pad_id, jnp.int32)]
    )
    wpad = -(-W.shape[0] // 8) * 8
    w_flat = jnp.concatenate(
        [W.reshape(-1), jnp.zeros((wpad - W.shape[0],), jnp.float32)]
    )

    f_table = _build_table_kernel(gpad, wpad, pad_id)(g2l_pad, w_flat)
    out = _aggregate_kernel(batch, rules_per_ex, gpad)(
        rules.reshape(-1), f_table
    )
    return out.reshape(batch, 1)
